# baseline (speedup 1.0000x reference)
import functools

import jax
import jax.numpy as jnp
from jax import lax
from jax.experimental import pallas as pl
from jax.experimental.pallas import tpu as pltpu
from jax.experimental.pallas import tpu_sc as plsc

N_SRC = 100000
N_MID = 16000
N_DST = 4000
E0 = 256000
E1 = 64000
P = 4000
D = 256
H = 256
HH = 128
WB = 384
ZPAD = 1000

NC = 2
NS = 16
LANES = 16


def _proj_z1_body(x_ref, wp_ref, bp_ref, q1_ref, bq1_ref, za_ref, zb_ref):
    pid = pl.program_id(0)
    h = jnp.dot(x_ref[...], wp_ref[...], preferred_element_type=jnp.float32)
    h = h + bp_ref[...]
    z = jnp.dot(h, q1_ref[...], preferred_element_type=jnp.float32)
    z = jax.nn.relu(z + bq1_ref[...])
    z = jnp.where(pid >= N_SRC // 1000, 0.0, z)
    za_ref[...] = z[:, :HH]
    zb_ref[...] = z[:, HH:]


def _proj_z1(x, W_proj, b_proj, Q1, bq1):
    blk = 1000
    grid = N_SRC // blk + ZPAD // blk
    nb = N_SRC // blk
    return pl.pallas_call(
        _proj_z1_body,
        grid=(grid,),
        in_specs=[
            pl.BlockSpec((blk, D), lambda i: (jnp.minimum(i, nb - 1), 0)),
            pl.BlockSpec((D, H), lambda i: (0, 0)),
            pl.BlockSpec((1, H), lambda i: (0, 0)),
            pl.BlockSpec((H, H), lambda i: (0, 0)),
            pl.BlockSpec((1, H), lambda i: (0, 0)),
        ],
        out_specs=[
            pl.BlockSpec((blk, HH), lambda i: (i, 0)),
            pl.BlockSpec((blk, HH), lambda i: (i, 0)),
        ],
        out_shape=[
            jax.ShapeDtypeStruct((N_SRC + ZPAD, HH), jnp.float32),
            jax.ShapeDtypeStruct((N_SRC + ZPAD, HH), jnp.float32),
        ],
    )(x, W_proj, b_proj.reshape(1, H), Q1, bq1.reshape(1, H))


def _layer1_body(x_ref, sa_ref, sb_ref, d_ref, wp_ref, bp_ref, w1a_ref,
                 w1b_ref, bw1_ref, q2_ref, bq2_ref, h1_ref, za_ref, zb_ref):
    dd = jnp.clip(d_ref[...], 1.0, None)
    agg = jnp.concatenate([sa_ref[...], sb_ref[...]], axis=1) / dd
    h_item = jnp.dot(x_ref[...], wp_ref[...],
                     preferred_element_type=jnp.float32) + bp_ref[...]
    h = jnp.dot(h_item, w1a_ref[...], preferred_element_type=jnp.float32)
    h = h + jnp.dot(agg, w1b_ref[...], preferred_element_type=jnp.float32)
    h = jax.nn.relu(h + bw1_ref[...])
    nrm = jnp.sqrt(jnp.sum(h * h, axis=1, keepdims=True))
    h = h / jnp.clip(nrm, 1e-6, None)
    h1_ref[...] = h
    z = jnp.dot(h, q2_ref[...], preferred_element_type=jnp.float32)
    z = jax.nn.relu(z + bq2_ref[...])
    za_ref[...] = z[:, :HH]
    zb_ref[...] = z[:, HH:]


def _layer1(x16, s1a, s1b, deg1, W_proj, b_proj, W1, bw1, Q2, bq2):
    blk = 1000
    grid = N_MID // blk
    return pl.pallas_call(
        _layer1_body,
        grid=(grid,),
        in_specs=[
            pl.BlockSpec((blk, D), lambda i: (i, 0)),
            pl.BlockSpec((blk, HH), lambda i: (i, 0)),
            pl.BlockSpec((blk, HH), lambda i: (i, 0)),
            pl.BlockSpec((blk, 1), lambda i: (i, 0)),
            pl.BlockSpec((D, H), lambda i: (0, 0)),
            pl.BlockSpec((1, H), lambda i: (0, 0)),
            pl.BlockSpec((H, H), lambda i: (0, 0)),
            pl.BlockSpec((H, H), lambda i: (0, 0)),
            pl.BlockSpec((1, H), lambda i: (0, 0)),
            pl.BlockSpec((H, H), lambda i: (0, 0)),
            pl.BlockSpec((1, H), lambda i: (0, 0)),
        ],
        out_specs=[
            pl.BlockSpec((blk, H), lambda i: (i, 0)),
            pl.BlockSpec((blk, HH), lambda i: (i, 0)),
            pl.BlockSpec((blk, HH), lambda i: (i, 0)),
        ],
        out_shape=[
            jax.ShapeDtypeStruct((N_MID, H), jnp.float32),
            jax.ShapeDtypeStruct((N_MID, HH), jnp.float32),
            jax.ShapeDtypeStruct((N_MID, HH), jnp.float32),
        ],
    )(x16, s1a, s1b, deg1, W_proj, b_proj.reshape(1, H), W1[:H], W1[H:],
      bw1.reshape(1, H), Q2, bq2.reshape(1, H))


def _layer2_body(h14_ref, la0_ref, la1_ref, rb0_ref, rb1_ref, d_ref, x4_ref,
                 wp_ref, bp_ref, w2a_ref, w2b_ref, bw2_ref, bias_ref, g_ref,
                 b_ref, hb_ref):
    dd = jnp.clip(d_ref[...], 1.0, None)
    agg = jnp.concatenate([la0_ref[...] + la1_ref[...],
                           rb0_ref[...] + rb1_ref[...]], axis=1) / dd
    h = jnp.dot(h14_ref[...], w2a_ref[...], preferred_element_type=jnp.float32)
    h = h + jnp.dot(agg, w2b_ref[...], preferred_element_type=jnp.float32)
    h = jax.nn.relu(h + bw2_ref[...])
    nrm = jnp.sqrt(jnp.sum(h * h, axis=1, keepdims=True))
    h = h / jnp.clip(nrm, 1e-6, None)
    hd = jnp.dot(x4_ref[...], wp_ref[...],
                 preferred_element_type=jnp.float32) + bp_ref[...]
    h = hd + h
    mu = jnp.mean(h, axis=1, keepdims=True)
    var = jnp.mean((h - mu) ** 2, axis=1, keepdims=True)
    h = (h - mu) / jnp.sqrt(var + 1e-5) * g_ref[...] + b_ref[...]
    hb_ref[...] = jnp.concatenate(
        [h, bias_ref[...], jnp.zeros((N_DST, WB - H - 1), jnp.float32)],
        axis=1)


def _layer2(h1_4, la0, la1, rb0, rb1, deg2, x4, W_proj, b_proj, W2, bw2,
            item_bias, gamma, beta):
    return pl.pallas_call(
        _layer2_body,
        out_shape=jax.ShapeDtypeStruct((N_DST, WB), jnp.float32),
    )(h1_4, la0, la1, rb0, rb1, deg2, x4, W_proj, b_proj.reshape(1, H),
      W2[:H], W2[H:], bw2.reshape(1, H), item_bias.reshape(N_DST, 1),
      gamma.reshape(1, H), beta.reshape(1, H))


def _score_body(u_ref, v_ref, nu_ref, nv_ref, pos_ref, neg_ref, loss_ref,
                auc_ref):
    u, v = u_ref[...], v_ref[...]
    nu, nv = nu_ref[...], nv_ref[...]
    pos = jnp.sum(u[:, :H] * v[:, :H], axis=1, keepdims=True)
    pos = pos + u[:, H:H + 1] + v[:, H:H + 1]
    neg = jnp.sum(nu[:, :H] * nv[:, :H], axis=1, keepdims=True)
    neg = neg + nu[:, H:H + 1] + nv[:, H:H + 1]
    pos_ref[...] = pos
    neg_ref[...] = neg
    loss_ref[...] = jnp.mean(jax.nn.relu(neg - pos + 1.0)).reshape(1, 1)
    auc_ref[...] = jnp.mean((pos > neg).astype(jnp.float32)).reshape(1, 1)


def _score(u, v, nu, nv):
    return pl.pallas_call(
        _score_body,
        out_shape=[
            jax.ShapeDtypeStruct((P, 1), jnp.float32),
            jax.ShapeDtypeStruct((P, 1), jnp.float32),
            jax.ShapeDtypeStruct((1, 1), jnp.float32),
            jax.ShapeDtypeStruct((1, 1), jnp.float32),
        ],
    )(u, v, nu, nv)


def _make_segsum(n_dst, edges_per_tile, chunk, dst_split, zrow=0,
                 with_deg=False):
    nchunks = edges_per_tile // chunk
    half = n_dst // NC if dst_split else n_dst
    out_rows = n_dst if dst_split else NC * n_dst
    nfull = half // 128
    rem = half - nfull * 128
    K = (nfull + (1 if rem else 0) + NS - 1) // NS

    mesh = plsc.VectorSubcoreMesh(core_axis_name="c", subcore_axis_name="s")

    out_type = [
        jax.ShapeDtypeStruct((out_rows, HH), jnp.float32),
        jax.ShapeDtypeStruct((out_rows, HH), jnp.float32),
    ]
    scratch = [
        pltpu.VMEM((chunk,), jnp.int32),
        pltpu.VMEM((chunk,), jnp.int32),
        pltpu.VMEM((chunk, HH), jnp.float32),
        pltpu.VMEM((128, HH), jnp.float32),
        pltpu.VMEM_SHARED((half, HH), jnp.float32),
        pltpu.SemaphoreType.DMA,
    ]
    if with_deg:
        out_type += [
            jax.ShapeDtypeStruct((n_dst, HH), jnp.float32),
            jax.ShapeDtypeStruct((N_DST, HH), jnp.float32),
        ]
        scratch += [
            pltpu.VMEM((80,), jnp.int32),
            pltpu.VMEM((80,), jnp.int32),
            pltpu.VMEM((80, HH), jnp.float32),
        ]

    @functools.partial(pl.kernel, mesh=mesh, out_type=out_type,
                       scratch_types=scratch)
    def seg(*args):
        if with_deg:
            (ta, tb, esrc, edst, ct, e1d, oa, ob, od1, od2,
             src_v, dst_v, rows_v, zero_v, acc_sh, sem,
             src2_v, dst2_v, rows2_v) = args
        else:
            (ta, tb, esrc, edst, oa, ob,
             src_v, dst_v, rows_v, zero_v, acc_sh, sem) = args
        c = lax.axis_index("c")
        s = lax.axis_index("s")
        _zero_vmem_2d(zero_v, 128, HH)

        def for_blocks(fn, nf=nfull, rm=rem, kk=K):
            for k in range(kk):
                if (k + 1) * NS <= nf:
                    fn(s + NS * k, 128)
                else:
                    @pl.when(s + NS * k < nf)
                    def _():
                        fn(s + NS * k, 128)
                    if rm:
                        @pl.when(s + NS * k == nf)
                        def _():
                            fn(nf, rm)

        def zero_acc():
            for_blocks(lambda b, sz: pltpu.sync_copy(
                zero_v.at[pl.ds(0, sz)], acc_sh.at[pl.ds(b * 128, sz)]))

        if dst_split:
            ebase = s * edges_per_tile
            base = c * half
        else:
            ebase = (c * NS + s) * edges_per_tile

        def run_pass(tbl, redirect):
            def chunk_body(j, _):
                eoff = ebase + j * chunk
                pltpu.sync_copy(esrc.at[pl.ds(eoff, chunk)], src_v)
                pltpu.sync_copy(edst.at[pl.ds(eoff, chunk)], dst_v)
                if dst_split:
                    for k in range(chunk // LANES):
                        sl = pl.ds(k * LANES, LANES)
                        d = dst_v[sl]
                        local = d - base
                        inr = (local >= 0) & (local < half)
                        dst_v[sl] = jnp.where(inr, local, 0)
                        if redirect == "zrow":
                            src_v[sl] = jnp.where(inr, src_v[sl], zrow)
                        else:
                            src_v[sl] = jnp.where(inr, 0, 8)
                elif redirect == "ones":
                    for k in range(chunk // LANES):
                        src_v[pl.ds(k * LANES, LANES)] = jnp.zeros(
                            (LANES,), jnp.int32)
                pltpu.async_copy(tbl.at[src_v], rows_v, sem).wait()
                pltpu.sync_copy(rows_v, acc_sh.at[dst_v], add=True)
                return 0

            lax.fori_loop(0, nchunks, chunk_body, 0)

        def write_out(out, obase, nf=nfull, rm=rem, kk=K):
            for_blocks(lambda b, sz: pltpu.sync_copy(
                acc_sh.at[pl.ds(b * 128, sz)],
                out.at[pl.ds(obase + b * 128, sz)]), nf, rm, kk)

        obase = c * half if dst_split else c * n_dst
        passes = [(ta, "zrow", oa), (tb, "zrow", ob)]
        if with_deg:
            passes.append((ct, "deg", od1))
        for i, (tbl, kind, out) in enumerate(passes):
            zero_acc()
            plsc.subcore_barrier()
            run_pass(tbl, kind)
            plsc.subcore_barrier()
            write_out(out, obase)
            if with_deg or i < len(passes) - 1:
                plsc.subcore_barrier()

        if with_deg:
            zero_acc()
            plsc.subcore_barrier()

            @pl.when(c == 0)
            def _():
                def chunk_body(j, _):
                    eoff = s * (E1 // NS) + j * 80
                    pltpu.sync_copy(e1d.at[pl.ds(eoff, 80)], dst2_v)

                    def fill(k, _):
                        src2_v[pl.ds(k * LANES, LANES)] = jnp.zeros(
                            (LANES,), jnp.int32)
                        return 0
                    lax.fori_loop(0, 80 // LANES, fill, 0)
                    pltpu.async_copy(ct.at[src2_v], rows2_v, sem).wait()
                    pltpu.sync_copy(rows2_v, acc_sh.at[dst2_v], add=True)
                    return 0

                lax.fori_loop(0, E1 // NS // 80, chunk_body, 0)

            plsc.subcore_barrier()

            @pl.when(c == 0)
            def _():
                nf2, rm2 = N_DST // 128, N_DST - (N_DST // 128) * 128
                kk2 = (nf2 + (1 if rm2 else 0) + NS - 1) // NS
                write_out(od2, 0, nf2, rm2, kk2)

    return seg


def _zero_vmem_2d(ref, rows, cols):
    zero = jnp.zeros((LANES,), jnp.float32)
    nj = cols // LANES

    def body(k, _):
        ref[k // nj, pl.ds((k % nj) * LANES, LANES)] = zero
        return 0

    lax.fori_loop(0, rows * nj, body, 0)


_segsum1 = _make_segsum(N_MID, E0 // NS, 128, dst_split=True,
                        zrow=N_SRC, with_deg=True)
_segsum2 = _make_segsum(N_DST, E1 // (NC * NS), 80, dst_split=False)


def _make_score_gather():
    mesh = plsc.VectorSubcoreMesh(core_axis_name="c", subcore_axis_name="s")
    PP = 4096
    per_tile = PP // (NC * NS)

    @functools.partial(
        pl.kernel,
        mesh=mesh,
        out_type=[jax.ShapeDtypeStruct((PP, WB), jnp.float32)
                  for _ in range(4)],
        scratch_types=[
            pltpu.VMEM((per_tile,), jnp.int32),
            pltpu.VMEM((per_tile, WB), jnp.float32),
            pltpu.SemaphoreType.DMA,
        ],
    )
    def gather4(hb, iu, iv, inu, inv, ou, ov, onu, onv, idx_v, rows_v, sem):
        c = lax.axis_index("c")
        s = lax.axis_index("s")
        wid = s * NC + c
        for inp, outp in ((iu, ou), (iv, ov), (inu, onu), (inv, onv)):
            pltpu.sync_copy(inp.at[pl.ds(wid * per_tile, per_tile)], idx_v)
            pltpu.async_copy(hb.at[idx_v], rows_v, sem).wait()
            pltpu.sync_copy(rows_v, outp.at[pl.ds(wid * per_tile, per_tile)])

    return gather4


_score_gather = _make_score_gather()


def _pad_idx(i):
    return jnp.concatenate([i, jnp.zeros((4096 - P,), jnp.int32)])


_USE_SC_SEG1 = True
_USE_SC_SEG2 = True
_USE_SC_GATHER = True


def kernel(x, e0_src, e0_dst, e1_src, e1_dst, pos_u, pos_v, neg_u, neg_v,
           W_proj, b_proj, Q1, bq1, W1, bw1, Q2, bq2, W2, bw2,
           item_bias, gamma, beta):
    z1a, z1b = _proj_z1(x, W_proj, b_proj, Q1, bq1)
    if _USE_SC_SEG1:
        ct = jnp.zeros((16, HH), jnp.float32).at[0].set(1.0)
        s1a, s1b, od1, od2 = _segsum1(z1a, z1b, e0_src, e0_dst, ct, e1_dst)
        deg1 = od1[:, :1]
        deg2 = od2[:, :1]
    else:
        s1a = jax.ops.segment_sum(z1a[e0_src], e0_dst, num_segments=N_MID)
        s1b = jax.ops.segment_sum(z1b[e0_src], e0_dst, num_segments=N_MID)
        deg1 = jax.ops.segment_sum(jnp.ones((E0, 1), jnp.float32), e0_dst,
                                   num_segments=N_MID)
        deg2 = jax.ops.segment_sum(jnp.ones((E1, 1), jnp.float32), e1_dst,
                                   num_segments=N_DST)
    h1, z2a, z2b = _layer1(x[:N_MID], s1a, s1b, deg1, W_proj, b_proj, W1,
                           bw1, Q2, bq2)
    if _USE_SC_SEG2:
        s2a, s2b = _segsum2(z2a, z2b, e1_src, e1_dst)
    else:
        s2a = jnp.concatenate([
            jax.ops.segment_sum(z2a[e1_src[:E1 // 2]], e1_dst[:E1 // 2],
                                num_segments=N_DST),
            jax.ops.segment_sum(z2a[e1_src[E1 // 2:]], e1_dst[E1 // 2:],
                                num_segments=N_DST)])
        s2b = jnp.concatenate([
            jax.ops.segment_sum(z2b[e1_src[:E1 // 2]], e1_dst[:E1 // 2],
                                num_segments=N_DST),
            jax.ops.segment_sum(z2b[e1_src[E1 // 2:]], e1_dst[E1 // 2:],
                                num_segments=N_DST)])
    hb = _layer2(h1[:N_DST], s2a[:N_DST], s2a[N_DST:], s2b[:N_DST],
                 s2b[N_DST:], deg2, x[:N_DST], W_proj, b_proj, W2, bw2,
                 item_bias, gamma, beta)
    if _USE_SC_GATHER:
        u, v, nu, nv = _score_gather(hb, _pad_idx(pos_u), _pad_idx(pos_v),
                                     _pad_idx(neg_u), _pad_idx(neg_v))
        u, v, nu, nv = u[:P], v[:P], nu[:P], nv[:P]
    else:
        u, v, nu, nv = hb[pos_u], hb[pos_v], hb[neg_u], hb[neg_v]
    pos, neg, loss, auc = _score(u, v, nu, nv)
    return (pos[:, 0], neg[:, 0], loss[0, 0], auc[0, 0])


# speedup vs baseline: 23.3858x; 23.3858x over previous
import functools

import jax
import jax.numpy as jnp
from jax import lax
from jax.experimental import pallas as pl
from jax.experimental.pallas import tpu as pltpu
from jax.experimental.pallas import tpu_sc as plsc

N_SRC = 100000
N_MID = 16000
N_DST = 4000
E0 = 256000
E1 = 64000
P = 4000
D = 256
H = 256
HH = 128
WB = 384
ZPAD = 1000

NC = 2
NS = 16
LANES = 16


def _proj_z1_body(x_ref, wp_ref, bp_ref, q1_ref, bq1_ref, za_ref, zb_ref):
    h = jnp.dot(x_ref[...], wp_ref[...], preferred_element_type=jnp.float32)
    h = h + bp_ref[...]
    z = jnp.dot(h, q1_ref[...], preferred_element_type=jnp.float32)
    z = jax.nn.relu(z + bq1_ref[...])
    za_ref[...] = z[:, :HH]
    zb_ref[...] = z[:, HH:]


def _proj_z1(x, W_proj, b_proj, Q1, bq1):
    blk = 1000
    grid = N_SRC // blk
    return pl.pallas_call(
        _proj_z1_body,
        grid=(grid,),
        in_specs=[
            pl.BlockSpec((blk, D), lambda i: (i, 0)),
            pl.BlockSpec((D, H), lambda i: (0, 0)),
            pl.BlockSpec((1, H), lambda i: (0, 0)),
            pl.BlockSpec((H, H), lambda i: (0, 0)),
            pl.BlockSpec((1, H), lambda i: (0, 0)),
        ],
        out_specs=[
            pl.BlockSpec((blk, HH), lambda i: (i, 0)),
            pl.BlockSpec((blk, HH), lambda i: (i, 0)),
        ],
        out_shape=[
            jax.ShapeDtypeStruct((N_SRC, HH), jnp.float32),
            jax.ShapeDtypeStruct((N_SRC, HH), jnp.float32),
        ],
    )(x, W_proj, b_proj.reshape(1, H), Q1, bq1.reshape(1, H))


def _layer1_body(x_ref, sa_ref, sb_ref, d_ref, wp_ref, bp_ref, w1a_ref,
                 w1b_ref, bw1_ref, q2_ref, bq2_ref, h1_ref, za_ref, zb_ref):
    dd = jnp.clip(d_ref[...], 1.0, None)
    agg = jnp.concatenate([sa_ref[...], sb_ref[...]], axis=1) / dd
    h_item = jnp.dot(x_ref[...], wp_ref[...],
                     preferred_element_type=jnp.float32) + bp_ref[...]
    h = jnp.dot(h_item, w1a_ref[...], preferred_element_type=jnp.float32)
    h = h + jnp.dot(agg, w1b_ref[...], preferred_element_type=jnp.float32)
    h = jax.nn.relu(h + bw1_ref[...])
    nrm = jnp.sqrt(jnp.sum(h * h, axis=1, keepdims=True))
    h = h / jnp.clip(nrm, 1e-6, None)
    h1_ref[...] = h
    z = jnp.dot(h, q2_ref[...], preferred_element_type=jnp.float32)
    z = jax.nn.relu(z + bq2_ref[...])
    za_ref[...] = z[:, :HH]
    zb_ref[...] = z[:, HH:]


def _layer1(x16, s1a, s1b, deg1, W_proj, b_proj, W1, bw1, Q2, bq2):
    blk = 1000
    grid = N_MID // blk
    return pl.pallas_call(
        _layer1_body,
        grid=(grid,),
        in_specs=[
            pl.BlockSpec((blk, D), lambda i: (i, 0)),
            pl.BlockSpec((blk, HH), lambda i: (i, 0)),
            pl.BlockSpec((blk, HH), lambda i: (i, 0)),
            pl.BlockSpec((blk, 1), lambda i: (i, 0)),
            pl.BlockSpec((D, H), lambda i: (0, 0)),
            pl.BlockSpec((1, H), lambda i: (0, 0)),
            pl.BlockSpec((H, H), lambda i: (0, 0)),
            pl.BlockSpec((H, H), lambda i: (0, 0)),
            pl.BlockSpec((1, H), lambda i: (0, 0)),
            pl.BlockSpec((H, H), lambda i: (0, 0)),
            pl.BlockSpec((1, H), lambda i: (0, 0)),
        ],
        out_specs=[
            pl.BlockSpec((blk, H), lambda i: (i, 0)),
            pl.BlockSpec((blk, HH), lambda i: (i, 0)),
            pl.BlockSpec((blk, HH), lambda i: (i, 0)),
        ],
        out_shape=[
            jax.ShapeDtypeStruct((N_MID, H), jnp.float32),
            jax.ShapeDtypeStruct((N_MID, HH), jnp.float32),
            jax.ShapeDtypeStruct((N_MID, HH), jnp.float32),
        ],
    )(x16, s1a, s1b, deg1, W_proj, b_proj.reshape(1, H), W1[:H], W1[H:],
      bw1.reshape(1, H), Q2, bq2.reshape(1, H))


def _layer2_body(h14_ref, sa_ref, sb_ref, d_ref, x4_ref,
                 wp_ref, bp_ref, w2a_ref, w2b_ref, bw2_ref, bias_ref, g_ref,
                 b_ref, hb_ref):
    dd = jnp.clip(d_ref[...], 1.0, None)
    agg = jnp.concatenate([sa_ref[...], sb_ref[...]], axis=1) / dd
    h = jnp.dot(h14_ref[...], w2a_ref[...], preferred_element_type=jnp.float32)
    h = h + jnp.dot(agg, w2b_ref[...], preferred_element_type=jnp.float32)
    h = jax.nn.relu(h + bw2_ref[...])
    nrm = jnp.sqrt(jnp.sum(h * h, axis=1, keepdims=True))
    h = h / jnp.clip(nrm, 1e-6, None)
    hd = jnp.dot(x4_ref[...], wp_ref[...],
                 preferred_element_type=jnp.float32) + bp_ref[...]
    h = hd + h
    mu = jnp.mean(h, axis=1, keepdims=True)
    var = jnp.mean((h - mu) ** 2, axis=1, keepdims=True)
    h = (h - mu) / jnp.sqrt(var + 1e-5) * g_ref[...] + b_ref[...]
    hb_ref[...] = jnp.concatenate(
        [h, bias_ref[...], jnp.zeros((N_DST, WB - H - 1), jnp.float32)],
        axis=1)


def _layer2(h1_4, s2a, s2b, deg2, x4, W_proj, b_proj, W2, bw2,
            item_bias, gamma, beta):
    return pl.pallas_call(
        _layer2_body,
        out_shape=jax.ShapeDtypeStruct((N_DST, WB), jnp.float32),
    )(h1_4, s2a, s2b, deg2, x4, W_proj, b_proj.reshape(1, H),
      W2[:H], W2[H:], bw2.reshape(1, H), item_bias.reshape(N_DST, 1),
      gamma.reshape(1, H), beta.reshape(1, H))


def _score_body(u_ref, v_ref, nu_ref, nv_ref, pos_ref, neg_ref, loss_ref,
                auc_ref):
    u, v = u_ref[...], v_ref[...]
    nu, nv = nu_ref[...], nv_ref[...]
    pos = jnp.sum(u[:, :H] * v[:, :H], axis=1, keepdims=True)
    pos = pos + u[:, H:H + 1] + v[:, H:H + 1]
    neg = jnp.sum(nu[:, :H] * nv[:, :H], axis=1, keepdims=True)
    neg = neg + nu[:, H:H + 1] + nv[:, H:H + 1]
    pos_ref[...] = pos
    neg_ref[...] = neg
    loss_ref[...] = jnp.mean(jax.nn.relu(neg - pos + 1.0)).reshape(1, 1)
    auc_ref[...] = jnp.mean((pos > neg).astype(jnp.float32)).reshape(1, 1)


def _score(u, v, nu, nv):
    return pl.pallas_call(
        _score_body,
        out_shape=[
            jax.ShapeDtypeStruct((P, 1), jnp.float32),
            jax.ShapeDtypeStruct((P, 1), jnp.float32),
            jax.ShapeDtypeStruct((1, 1), jnp.float32),
            jax.ShapeDtypeStruct((1, 1), jnp.float32),
        ],
    )(u, v, nu, nv)


def _zero_vmem_2d(ref, rows, cols):
    zero = jnp.zeros((LANES,), jnp.float32)
    nj = cols // LANES

    def body(k, _):
        ref[k // nj, pl.ds((k % nj) * LANES, LANES)] = zero
        return 0

    lax.fori_loop(0, rows * nj, body, 0)


def _make_segsum(n_dst, chunk, with_deg=False):
    G = 2
    half = n_dst // NC
    acc_rows = half + 8
    nfull = half // 128
    rem = half - nfull * 128
    K = (nfull + (1 if rem else 0) + NS - 1) // NS
    nfz = acc_rows // 128
    rmz = acc_rows - nfz * 128
    KZ = (nfz + (1 if rmz else 0) + NS - 1) // NS

    mesh = plsc.VectorSubcoreMesh(core_axis_name="c", subcore_axis_name="s")

    out_type = [
        jax.ShapeDtypeStruct((n_dst, HH), jnp.float32),
        jax.ShapeDtypeStruct((n_dst, HH), jnp.float32),
    ]
    scratch = (
        [pltpu.VMEM((G * chunk,), jnp.int32)]
        + [pltpu.VMEM((chunk,), jnp.int32) for _ in range(G)]
        + [pltpu.VMEM((G, chunk, HH), jnp.float32),
           pltpu.VMEM((chunk, HH), jnp.float32),
           pltpu.VMEM_SHARED((acc_rows, HH), jnp.float32),
           pltpu.SemaphoreType.DMA,
           pltpu.SemaphoreType.DMA,
           pltpu.SemaphoreType.DMA]
    )
    if with_deg:
        out_type += [
            jax.ShapeDtypeStruct((n_dst, HH), jnp.float32),
            jax.ShapeDtypeStruct((NC * N_DST, HH), jnp.float32),
        ]
        scratch += [pltpu.VMEM((40,), jnp.int32) for _ in range(G)]

    @functools.partial(pl.kernel, mesh=mesh, out_type=out_type,
                       scratch_types=scratch)
    def seg(*args):
        if with_deg:
            (ta, tb, esrc, edst, zin, e1d, oa, ob, od1, od2,
             src_v, *rest) = args
        else:
            (ta, tb, esrc, edst, zin, oa, ob, src_v, *rest) = args
        dst_v = rest[:G]
        (rows_v, ones_v, acc_sh, isem, gsem, ssem) = rest[G:G + 6]
        d40_v = rest[G + 6:]
        c = lax.axis_index("c")
        s = lax.axis_index("s")
        one = jnp.full((LANES,), 1.0, jnp.float32)

        def fill(k, _):
            ones_v[k // (HH // LANES),
                   pl.ds((k % (HH // LANES)) * LANES, LANES)] = one
            return 0
        lax.fori_loop(0, chunk * (HH // LANES), fill, 0)

        def for_blocks(fn, nf, rm, kk):
            for k in range(kk):
                if (k + 1) * NS <= nf:
                    fn(s + NS * k, 128)
                else:
                    @pl.when(s + NS * k < nf)
                    def _():
                        fn(s + NS * k, 128)
                    if rm:
                        @pl.when(s + NS * k == nf)
                        def _():
                            fn(nf, rm)

        def zero_acc():
            for_blocks(lambda b, sz: pltpu.sync_copy(
                zin.at[pl.ds(0, sz)], acc_sh.at[pl.ds(b * 128, sz)]),
                nfz, rmz, KZ)

        def write_out(out, obase):
            for_blocks(lambda b, sz: pltpu.sync_copy(
                acc_sh.at[pl.ds(b * 128, sz)],
                out.at[pl.ds(obase + b * 128, sz)]), nfull, rem, K)

        base = c * half

        def fixup(bufs, chunk2, b):
            for k in range(chunk2 // LANES):
                sl = pl.ds(k * LANES, LANES)
                d = bufs[b][sl]
                local = d - base
                inr = (local >= 0) & (local < half)
                bufs[b][sl] = jnp.where(inr, local, half)

        def run_pass(tbl, esrc2, edst2, ept, out, obase):
            zero_acc()
            plsc.subcore_barrier()
            ebase = s * ept

            def group(g, _):
                eoff = ebase + g * G * chunk
                ds_ = [pltpu.async_copy(
                    esrc2.at[pl.ds(eoff, G * chunk)], src_v, isem)]
                ds_ += [pltpu.async_copy(
                    edst2.at[pl.ds(eoff + b * chunk, chunk)], dst_v[b],
                    isem) for b in range(G)]
                for d in ds_:
                    d.wait()
                for b in range(G):
                    fixup(dst_v, chunk, b)
                gs = [pltpu.async_copy(
                    tbl.at[src_v.at[pl.ds(b * chunk, chunk)]],
                    rows_v.at[b], gsem) for b in range(G)]
                for d in gs:
                    d.wait()
                ss = [pltpu.async_copy(
                    rows_v.at[b], acc_sh.at[dst_v[b]], ssem, add=True)
                    for b in range(G)]
                for d in ss:
                    d.wait()
                return 0

            lax.fori_loop(0, ept // (G * chunk), group, 0)
            plsc.subcore_barrier()
            write_out(out, obase)

        def deg_pass(edges, chunk2, ept, out, obase, split, bufs,
                     wo=None):
            zero_acc()
            plsc.subcore_barrier()
            ebase = (s if split else c * NS + s) * ept

            def group(g, _):
                eoff = ebase + g * G * chunk2
                ds_ = [pltpu.async_copy(
                    edges.at[pl.ds(eoff + b * chunk2, chunk2)], bufs[b],
                    isem) for b in range(G)]
                for d in ds_:
                    d.wait()
                if split:
                    for b in range(G):
                        fixup(bufs, chunk2, b)
                ss = [pltpu.async_copy(
                    ones_v.at[pl.ds(0, chunk2)], acc_sh.at[bufs[b]],
                    ssem, add=True) for b in range(G)]
                for d in ss:
                    d.wait()
                return 0

            lax.fori_loop(0, ept // (G * chunk2), group, 0)
            plsc.subcore_barrier()
            if wo is None:
                write_out(out, obase)
            else:
                for_blocks(lambda b, sz: pltpu.sync_copy(
                    acc_sh.at[pl.ds(b * 128, sz)],
                    out.at[pl.ds(obase + b * 128, sz)]), *wo)

        obase = c * half
        run_pass(ta, esrc, edst, E0 // NS if with_deg else E1 // NS,
                 oa, obase)
        plsc.subcore_barrier()
        run_pass(tb, esrc, edst, E0 // NS if with_deg else E1 // NS,
                 ob, obase)

        if with_deg:
            plsc.subcore_barrier()
            deg_pass(edst, chunk, E0 // NS, od1, obase, True, dst_v)
            plsc.subcore_barrier()
            deg_pass(e1d, 40, E1 // (NC * NS), od2, c * N_DST, False,
                     d40_v, wo=(N_DST // 128, N_DST % 128,
                                (N_DST // 128 + 1 + NS - 1) // NS))

    return seg


_segsum1 = _make_segsum(N_MID, 64, with_deg=True)
_segsum2 = _make_segsum(N_DST, 40)


def _make_score_gather():
    mesh = plsc.VectorSubcoreMesh(core_axis_name="c", subcore_axis_name="s")
    PP = 4096
    per_tile = PP // (NC * NS)

    @functools.partial(
        pl.kernel,
        mesh=mesh,
        out_type=[jax.ShapeDtypeStruct((PP, WB), jnp.float32)
                  for _ in range(4)],
        scratch_types=[
            pltpu.VMEM((64,), jnp.int32),
            pltpu.VMEM((64, WB), jnp.float32),
            pltpu.SemaphoreType.DMA,
        ],
    )
    def gather4(hb, iu, iv, inu, inv, ou, ov, onu, onv, idx_v, rows_v, sem):
        c = lax.axis_index("c")
        s = lax.axis_index("s")
        wid = s * NC + c
        for inp, outp in ((iu, ou), (iv, ov), (inu, onu), (inv, onv)):
            for k in range(per_tile // 64):
                off = wid * per_tile + k * 64
                pltpu.sync_copy(inp.at[pl.ds(off, 64)], idx_v)
                pltpu.async_copy(hb.at[idx_v], rows_v, sem).wait()
                pltpu.sync_copy(rows_v, outp.at[pl.ds(off, 64)])

    return gather4


_score_gather = _make_score_gather()


def _pad_idx(i):
    return jnp.concatenate([i, jnp.zeros((4096 - P,), jnp.int32)])


_USE_SC_SEG1 = True
_USE_SC_SEG2 = True
_USE_SC_GATHER = True


def kernel(x, e0_src, e0_dst, e1_src, e1_dst, pos_u, pos_v, neg_u, neg_v,
           W_proj, b_proj, Q1, bq1, W1, bw1, Q2, bq2, W2, bw2,
           item_bias, gamma, beta):
    z1a, z1b = _proj_z1(x, W_proj, b_proj, Q1, bq1)
    zin = jnp.zeros((128, HH), jnp.float32)
    if _USE_SC_SEG1:
        s1a, s1b, od1, od2 = _segsum1(z1a, z1b, e0_src, e0_dst, zin, e1_dst)
        deg1 = od1[:, :1]
        deg2 = od2[:N_DST, :1] + od2[N_DST:, :1]
    else:
        s1a = jax.ops.segment_sum(z1a[e0_src], e0_dst, num_segments=N_MID)
        s1b = jax.ops.segment_sum(z1b[e0_src], e0_dst, num_segments=N_MID)
        deg1 = jax.ops.segment_sum(jnp.ones((E0, 1), jnp.float32), e0_dst,
                                   num_segments=N_MID)
        deg2 = jax.ops.segment_sum(jnp.ones((E1, 1), jnp.float32), e1_dst,
                                   num_segments=N_DST)
    h1, z2a, z2b = _layer1(x[:N_MID], s1a, s1b, deg1, W_proj, b_proj, W1,
                           bw1, Q2, bq2)
    if _USE_SC_SEG2:
        s2a, s2b = _segsum2(z2a, z2b, e1_src, e1_dst, zin)
    else:
        s2a = jax.ops.segment_sum(z2a[e1_src], e1_dst, num_segments=N_DST)
        s2b = jax.ops.segment_sum(z2b[e1_src], e1_dst, num_segments=N_DST)
    hb = _layer2(h1[:N_DST], s2a, s2b, deg2, x[:N_DST], W_proj, b_proj,
                 W2, bw2, item_bias, gamma, beta)
    if _USE_SC_GATHER:
        u, v, nu, nv = _score_gather(hb, _pad_idx(pos_u), _pad_idx(pos_v),
                                     _pad_idx(neg_u), _pad_idx(neg_v))
        u, v, nu, nv = u[:P], v[:P], nu[:P], nv[:P]
    else:
        u, v, nu, nv = hb[pos_u], hb[pos_v], hb[neg_u], hb[neg_v]
    pos, neg, loss, auc = _score(u, v, nu, nv)
    return (pos[:, 0], neg[:, 0], loss[0, 0], auc[0, 0])
